# baseline (speedup 1.0000x reference)
import functools

import jax
import jax.numpy as jnp
from jax import lax
from jax.experimental import pallas as pl
from jax.experimental.pallas import tpu as pltpu

_ANY_VMEM = pl.BlockSpec(memory_space=pltpu.MemorySpace.VMEM)
_BIG = 1e30
_SF_NUM = 64


def _ceil_to(v, m):
    return (v + m - 1) // m * m


def _adj_body(p_ref, adj_ref, scr_ref, *, chunk, nb, nchunks):
    @pl.when(pl.program_id(0) == 0)
    def _():
        adj_ref[...] = jnp.zeros_like(adj_ref)
        scr_ref[...] = jnp.zeros_like(scr_ref)

    sub = lax.broadcasted_iota(jnp.int8, (1, nb, 128), 1)
    lane = lax.broadcasted_iota(jnp.int8, (1, nb, 128), 2)
    one = jnp.ones((1, nb, 128), jnp.int8)
    zero = jnp.zeros((1, nb, 128), jnp.int8)
    unroll = 8

    def body(t, carry):
        base = t * (2 * unroll)
        ds, ohs = [], []
        for u in range(2 * unroll):
            e = base + u
            p = p_ref[0, 0, e]
            ds.append(jnp.right_shift(p, 13))
            k = jnp.bitwise_and(jnp.right_shift(p, 7), 63).astype(jnp.int8)
            l = jnp.bitwise_and(p, 127).astype(jnp.int8)
            ohs.append(jnp.where((sub == k) & (lane == l), one, zero))
        for half, ref in ((0, adj_ref), (1, scr_ref)):
            o = half * unroll
            rows = []
            for u in range(unroll):
                rows.append(ref[pl.ds(ds[o + u], 1), :, :])
            for j in range(1, unroll):
                for i in range(j):
                    ohs[o + j] = jnp.where(ds[o + i] == ds[o + j],
                                           ohs[o + j] | ohs[o + i], ohs[o + j])
            for u in range(unroll):
                ref[pl.ds(ds[o + u], 1), :, :] = rows[u] | ohs[o + u]
        return carry

    lax.fori_loop(0, chunk // (2 * unroll), body, 0)

    @pl.when(pl.program_id(0) == nchunks - 1)
    def _():
        adj_ref[...] = adj_ref[...] | scr_ref[...]


def _proj_body(x_ref, w1_ref, c1T_ref, a1d_ref, xp1_ref, alsT_ref, ald_ref):
    f32 = jnp.float32
    xp1 = jnp.dot(x_ref[...], w1_ref[...], preferred_element_type=f32)
    xp1_ref[...] = xp1.astype(xp1_ref.dtype)
    ald_ref[...] = jnp.dot(xp1.astype(a1d_ref.dtype), a1d_ref[...],
                           preferred_element_type=f32)
    alsT_ref[...] = lax.dot_general(
        c1T_ref[...], x_ref[...],
        dimension_numbers=(((1,), (1,)), ((), ())),
        preferred_element_type=f32)


def _edge_mask(adj_ref, tile_m):
    n = adj_ref.shape[1]
    r0 = pl.program_id(0) * tile_m
    col = lax.broadcasted_iota(jnp.int32, (tile_m, n), 1)
    row = lax.broadcasted_iota(jnp.int32, (tile_m, n), 0) + r0
    return (adj_ref[...] != 0) | (col == row)


def _attn1_body(adj_ref, xp1_ref, alsT_ref, ald_ref, b1_ref, w2_ref, xp2_ref,
                *, heads, hidden, tile_m):
    f32 = jnp.float32
    on_edge = _edge_mask(adj_ref, tile_m)
    ald = ald_ref[...]

    parts = []
    for h in range(heads):
        s = ald[:, h:h + 1] + alsT_ref[h:h + 1, :]
        s = jnp.maximum(s, 0.2 * s)
        z = jnp.where(on_edge, s, -_BIG)
        z = z - jnp.max(z, axis=1, keepdims=True)
        p = jnp.exp(z)
        den = jnp.sum(p, axis=1, keepdims=True)
        attn = p * pl.reciprocal(den, approx=True)
        agg = jnp.dot(attn.astype(xp1_ref.dtype),
                      xp1_ref[:, h * hidden:(h + 1) * hidden],
                      preferred_element_type=f32)
        parts.append(jnp.maximum(agg + b1_ref[0:1, h * hidden:(h + 1) * hidden],
                                 0.0))
    h1 = jnp.concatenate(parts, axis=1).astype(w2_ref.dtype)
    xp2_ref[...] = jnp.dot(h1, w2_ref[...],
                           preferred_element_type=f32).astype(xp2_ref.dtype)


def _attn2_body(adj_ref, xp2_ref, a2sT_ref, a2d_ref, b2_ref, o_ref,
                *, classes, sf_num, tile_m):
    f32 = jnp.float32
    cp = o_ref.shape[1]
    on_edge = _edge_mask(adj_ref, tile_m)
    r0 = pl.multiple_of(pl.program_id(0) * tile_m, tile_m)
    xp2_rows = xp2_ref[pl.ds(r0, tile_m), :]
    ald = jnp.dot(xp2_rows, a2d_ref[...], preferred_element_type=f32)
    als = lax.dot_general(a2sT_ref[...], xp2_ref[...],
                          dimension_numbers=(((1,), (1,)), ((), ())),
                          preferred_element_type=f32)
    s = ald + als
    s = jnp.maximum(s, 0.2 * s)
    z = jnp.where(on_edge, s, -_BIG)
    z = z - jnp.max(z, axis=1, keepdims=True)
    p = jnp.exp(z)
    den = jnp.sum(p, axis=1, keepdims=True)
    attn = p * pl.reciprocal(den, approx=True)
    h2 = jnp.dot(attn.astype(xp2_ref.dtype), xp2_ref[...],
                 preferred_element_type=f32) + b2_ref[...]

    col = lax.broadcasted_iota(jnp.int32, (tile_m, cp), 1)

    def seg_softmax(seg):
        zz = jnp.where(seg, h2, -_BIG)
        zz = zz - jnp.max(zz, axis=1, keepdims=True)
        pz = jnp.exp(zz) * seg.astype(f32)
        dd = jnp.maximum(jnp.sum(pz, axis=1, keepdims=True), 1e-30)
        return pz * pl.reciprocal(dd, approx=True)

    o_ref[...] = (seg_softmax(col < sf_num)
                  + seg_softmax(jnp.logical_and(col >= sf_num, col < classes)))


@functools.partial(jax.jit, static_argnames=("heads", "hidden", "classes",
                                             "sf_num", "tile_m"))
def _gat(x, edge_index, w1, asrc1, adst1, b1, w2, asrc2, adst2, b2,
         *, heads, hidden, classes, sf_num, tile_m):
    n, f = x.shape
    hd1 = heads * hidden
    tm = tile_m
    np_ = _ceil_to(n, tm)
    fp = _ceil_to(f, 128)
    cp = _ceil_to(classes, 128)
    hp = _ceil_to(heads, 8)
    hd1p = _ceil_to(hd1, 128)
    bf16, f32 = jnp.bfloat16, jnp.float32
    grid = (np_ // tm,)
    cparams = pltpu.CompilerParams(
        dimension_semantics=("parallel",),
        vmem_limit_bytes=48 * 1024 * 1024)

    x_pad = jnp.zeros((np_, fp), f32).at[:n, :f].set(x).astype(bf16)

    src, dst = edge_index[0], edge_index[1]
    n_edges = src.shape[0]
    nb = np_ // 128
    chunk = 2560
    ep = _ceil_to(n_edges, chunk)
    nchunks = ep // chunk
    p_e = jnp.full((ep,), 63 << 7, jnp.int32).at[:n_edges].set(
        jnp.left_shift(dst, 13)
        | jnp.left_shift(jnp.right_shift(src, 7), 7)
        | jnp.bitwise_and(src, 127))

    smem_spec = pl.BlockSpec((1, 1, chunk), lambda i: (i, 0, 0),
                             memory_space=pltpu.MemorySpace.SMEM)
    adj3 = pl.pallas_call(
        functools.partial(_adj_body, chunk=chunk, nb=nb, nchunks=nchunks),
        out_shape=jax.ShapeDtypeStruct((np_, nb, 128), jnp.int8),
        grid=(nchunks,),
        in_specs=[smem_spec],
        out_specs=pl.BlockSpec((np_, nb, 128), lambda i: (0, 0, 0)),
        scratch_shapes=[pltpu.VMEM((np_, nb, 128), jnp.int8)],
        compiler_params=pltpu.CompilerParams(
            dimension_semantics=("arbitrary",),
            vmem_limit_bytes=48 * 1024 * 1024),
    )(p_e.reshape(nchunks, 1, chunk))
    adj = adj3.reshape(np_, np_)

    w1_pad = jnp.zeros((fp, hd1p), f32).at[:f, :hd1].set(w1).astype(bf16)
    c1 = jnp.einsum("fhk,hk->hf", w1.reshape(f, heads, hidden), asrc1)
    c1T = jnp.zeros((hp, fp), f32).at[:heads, :f].set(c1).astype(bf16)
    a1d = (adst1[:, :, None] * jnp.eye(heads, dtype=f32)[:, None, :]
           ).reshape(hd1, heads)
    a1d = jnp.zeros((hd1p, hp), f32).at[:hd1, :heads].set(a1d).astype(bf16)
    b1p = jnp.zeros((1, hd1p), f32).at[0, :hd1].set(b1[0])

    w2_pad = jnp.zeros((hd1p, cp), f32).at[:hd1, :classes].set(w2).astype(bf16)
    a2sT = jnp.zeros((1, cp), f32).at[0, :classes].set(asrc2[0]).astype(bf16)
    a2d = jnp.zeros((cp, 1), f32).at[:classes, 0].set(adst2[0]).astype(bf16)
    b2p = jnp.zeros((1, cp), f32).at[0, :classes].set(b2[0])

    xp1, alsT, ald = pl.pallas_call(
        _proj_body,
        out_shape=(jax.ShapeDtypeStruct((np_, hd1p), bf16),
                   jax.ShapeDtypeStruct((hp, np_), f32),
                   jax.ShapeDtypeStruct((np_, hp), f32)),
        grid=grid,
        in_specs=[pl.BlockSpec((tm, fp), lambda i: (i, 0)),
                  _ANY_VMEM, _ANY_VMEM, _ANY_VMEM],
        out_specs=(pl.BlockSpec((tm, hd1p), lambda i: (i, 0)),
                   pl.BlockSpec((hp, tm), lambda i: (0, i)),
                   pl.BlockSpec((tm, hp), lambda i: (i, 0))),
        compiler_params=cparams,
    )(x_pad, w1_pad, c1T, a1d)

    xp2 = pl.pallas_call(
        functools.partial(_attn1_body, heads=heads, hidden=hidden, tile_m=tm),
        out_shape=jax.ShapeDtypeStruct((np_, cp), bf16),
        grid=grid,
        in_specs=[pl.BlockSpec((tm, np_), lambda i: (i, 0)),
                  _ANY_VMEM, _ANY_VMEM,
                  pl.BlockSpec((tm, hp), lambda i: (i, 0)),
                  _ANY_VMEM, _ANY_VMEM],
        out_specs=pl.BlockSpec((tm, cp), lambda i: (i, 0)),
        compiler_params=cparams,
    )(adj, xp1, alsT, ald, b1p, w2_pad)

    out = pl.pallas_call(
        functools.partial(_attn2_body, classes=classes, sf_num=sf_num,
                          tile_m=tm),
        out_shape=jax.ShapeDtypeStruct((np_, cp), f32),
        grid=grid,
        in_specs=[pl.BlockSpec((tm, np_), lambda i: (i, 0)),
                  _ANY_VMEM, _ANY_VMEM, _ANY_VMEM, _ANY_VMEM],
        out_specs=pl.BlockSpec((tm, cp), lambda i: (i, 0)),
        compiler_params=cparams,
    )(adj, xp2, a2sT, a2d, b2p)

    return out[:n, :sf_num], out[:n, sf_num:classes]


def kernel(x, edge_index, w1, asrc1, adst1, b1, w2, asrc2, adst2, b2):
    heads, hidden = asrc1.shape
    classes = w2.shape[1]
    return _gat(x, edge_index, w1, asrc1, adst1, b1, w2, asrc2, adst2, b2,
                heads=heads, hidden=hidden, classes=classes, sf_num=_SF_NUM,
                tile_m=256)


# speedup vs baseline: 1.1497x; 1.1497x over previous
import functools

import jax
import jax.numpy as jnp
from jax import lax
from jax.experimental import pallas as pl
from jax.experimental.pallas import tpu as pltpu

_ANY_VMEM = pl.BlockSpec(memory_space=pltpu.MemorySpace.VMEM)
_BIG = 1e30
_SF_NUM = 64


def _ceil_to(v, m):
    return (v + m - 1) // m * m


def _adj_body(d_ref, k_ref, l_ref, adj_ref, scr_ref, *, chunk, nb, nchunks):
    @pl.when(pl.program_id(0) == 0)
    def _():
        adj_ref[...] = jnp.zeros_like(adj_ref)
        scr_ref[...] = jnp.zeros_like(scr_ref)

    sub = lax.broadcasted_iota(jnp.int8, (1, nb, 128), 1)
    lane = lax.broadcasted_iota(jnp.int8, (1, nb, 128), 2)
    one = jnp.ones((1, nb, 128), jnp.int8)
    zero = jnp.zeros((1, nb, 128), jnp.int8)
    unroll = 4

    def body(t, carry):
        base = t * (2 * unroll)
        ds, ohs = [], []
        for u in range(2 * unroll):
            e = base + u
            ds.append(d_ref[0, 0, e])
            k = k_ref[0, 0, e].astype(jnp.int8)
            l = l_ref[0, 0, e].astype(jnp.int8)
            ohs.append(jnp.where((sub == k) & (lane == l), one, zero))
        for half, ref in ((0, adj_ref), (1, scr_ref)):
            o = half * unroll
            rows = []
            for u in range(unroll):
                rows.append(ref[pl.ds(ds[o + u], 1), :, :])
            for j in range(1, unroll):
                for i in range(j):
                    ohs[o + j] = jnp.where(ds[o + i] == ds[o + j],
                                           ohs[o + j] | ohs[o + i], ohs[o + j])
            for u in range(unroll):
                ref[pl.ds(ds[o + u], 1), :, :] = rows[u] | ohs[o + u]
        return carry

    lax.fori_loop(0, chunk // (2 * unroll), body, 0)

    @pl.when(pl.program_id(0) == nchunks - 1)
    def _():
        adj_ref[...] = adj_ref[...] | scr_ref[...]


def _proj_body(x_ref, w1_ref, c1T_ref, a1d_ref, xp1_ref, alsT_ref, ald_ref):
    f32 = jnp.float32
    xp1 = jnp.dot(x_ref[...], w1_ref[...], preferred_element_type=f32)
    xp1_ref[...] = xp1.astype(xp1_ref.dtype)
    ald_ref[...] = jnp.dot(xp1.astype(a1d_ref.dtype), a1d_ref[...],
                           preferred_element_type=f32)
    alsT_ref[...] = lax.dot_general(
        c1T_ref[...], x_ref[...],
        dimension_numbers=(((1,), (1,)), ((), ())),
        preferred_element_type=f32)


def _edge_mask(adj_ref, tile_m):
    n = adj_ref.shape[1]
    r0 = pl.program_id(0) * tile_m
    col = lax.broadcasted_iota(jnp.int32, (tile_m, n), 1)
    row = lax.broadcasted_iota(jnp.int32, (tile_m, n), 0) + r0
    return (adj_ref[...] != 0) | (col == row)


def _attn1_body(adj_ref, xp1_ref, alsT_ref, ald_ref, b1_ref, w2_ref, xp2_ref,
                *, heads, hidden, tile_m):
    f32 = jnp.float32
    on_edge = _edge_mask(adj_ref, tile_m)
    ald = ald_ref[...]

    parts = []
    for h in range(heads):
        s = ald[:, h:h + 1] + alsT_ref[h:h + 1, :]
        s = jnp.maximum(s, 0.2 * s)
        z = jnp.where(on_edge, s, -_BIG)
        z = z - jnp.max(z, axis=1, keepdims=True)
        p = jnp.exp(z)
        den = jnp.sum(p, axis=1, keepdims=True)
        agg = jnp.dot(p.astype(xp1_ref.dtype),
                      xp1_ref[:, h * hidden:(h + 1) * hidden],
                      preferred_element_type=f32)
        agg = agg * pl.reciprocal(den, approx=True)
        parts.append(jnp.maximum(agg + b1_ref[0:1, h * hidden:(h + 1) * hidden],
                                 0.0))
    h1 = jnp.concatenate(parts, axis=1).astype(w2_ref.dtype)
    xp2_ref[...] = jnp.dot(h1, w2_ref[...],
                           preferred_element_type=f32).astype(xp2_ref.dtype)


def _attn2_body(adj_ref, xp2_ref, a2sT_ref, a2d_ref, b2_ref, o_ref,
                *, classes, sf_num, tile_m):
    f32 = jnp.float32
    cp = o_ref.shape[1]
    on_edge = _edge_mask(adj_ref, tile_m)
    r0 = pl.multiple_of(pl.program_id(0) * tile_m, tile_m)
    xp2_rows = xp2_ref[pl.ds(r0, tile_m), :]
    ald = jnp.dot(xp2_rows, a2d_ref[...], preferred_element_type=f32)
    als = lax.dot_general(a2sT_ref[...], xp2_ref[...],
                          dimension_numbers=(((1,), (1,)), ((), ())),
                          preferred_element_type=f32)
    s = ald + als
    s = jnp.maximum(s, 0.2 * s)
    z = jnp.where(on_edge, s, -_BIG)
    z = z - jnp.max(z, axis=1, keepdims=True)
    p = jnp.exp(z)
    den = jnp.sum(p, axis=1, keepdims=True)
    h2 = jnp.dot(p.astype(xp2_ref.dtype), xp2_ref[...],
                 preferred_element_type=f32)
    h2 = h2 * pl.reciprocal(den, approx=True) + b2_ref[...]

    col = lax.broadcasted_iota(jnp.int32, (tile_m, cp), 1)

    def seg_softmax(seg):
        zz = jnp.where(seg, h2, -_BIG)
        zz = zz - jnp.max(zz, axis=1, keepdims=True)
        pz = jnp.exp(zz) * seg.astype(f32)
        dd = jnp.maximum(jnp.sum(pz, axis=1, keepdims=True), 1e-30)
        return pz * pl.reciprocal(dd, approx=True)

    o_ref[...] = (seg_softmax(col < sf_num)
                  + seg_softmax(jnp.logical_and(col >= sf_num, col < classes)))


@functools.partial(jax.jit, static_argnames=("heads", "hidden", "classes",
                                             "sf_num", "tile_m"))
def _gat(x, edge_index, w1, asrc1, adst1, b1, w2, asrc2, adst2, b2,
         *, heads, hidden, classes, sf_num, tile_m):
    n, f = x.shape
    hd1 = heads * hidden
    tm = tile_m
    np_ = _ceil_to(n, tm)
    fp = _ceil_to(f, 128)
    cp = _ceil_to(classes, 128)
    hp = _ceil_to(heads, 8)
    hd1p = _ceil_to(hd1, 128)
    bf16, f32 = jnp.bfloat16, jnp.float32
    grid = (np_ // tm,)
    cparams = pltpu.CompilerParams(
        dimension_semantics=("parallel",),
        vmem_limit_bytes=48 * 1024 * 1024)

    x_pad = jnp.zeros((np_, fp), f32).at[:n, :f].set(x).astype(bf16)

    src, dst = edge_index[0], edge_index[1]
    n_edges = src.shape[0]
    nb = np_ // 128
    chunk = 2560
    ep = _ceil_to(n_edges, chunk)
    nchunks = ep // chunk
    d_e = jnp.zeros((ep,), jnp.int32).at[:n_edges].set(dst)
    k_e = jnp.zeros((ep,), jnp.int32).at[:n_edges].set(jnp.right_shift(src, 7))
    l_e = jnp.full((ep,), -1, jnp.int32).at[:n_edges].set(
        jnp.bitwise_and(src, 127))

    smem_spec = pl.BlockSpec((1, 1, chunk), lambda i: (i, 0, 0),
                             memory_space=pltpu.MemorySpace.SMEM)
    adj3 = pl.pallas_call(
        functools.partial(_adj_body, chunk=chunk, nb=nb, nchunks=nchunks),
        out_shape=jax.ShapeDtypeStruct((np_, nb, 128), jnp.int8),
        grid=(nchunks,),
        in_specs=[smem_spec, smem_spec, smem_spec],
        out_specs=pl.BlockSpec((np_, nb, 128), lambda i: (0, 0, 0)),
        scratch_shapes=[pltpu.VMEM((np_, nb, 128), jnp.int8)],
        compiler_params=pltpu.CompilerParams(
            dimension_semantics=("arbitrary",),
            vmem_limit_bytes=48 * 1024 * 1024),
    )(d_e.reshape(nchunks, 1, chunk), k_e.reshape(nchunks, 1, chunk),
      l_e.reshape(nchunks, 1, chunk))
    adj = adj3.reshape(np_, np_)

    w1_pad = jnp.zeros((fp, hd1p), f32).at[:f, :hd1].set(w1).astype(bf16)
    c1 = jnp.einsum("fhk,hk->hf", w1.reshape(f, heads, hidden), asrc1)
    c1T = jnp.zeros((hp, fp), f32).at[:heads, :f].set(c1).astype(bf16)
    a1d = (adst1[:, :, None] * jnp.eye(heads, dtype=f32)[:, None, :]
           ).reshape(hd1, heads)
    a1d = jnp.zeros((hd1p, hp), f32).at[:hd1, :heads].set(a1d).astype(bf16)
    b1p = jnp.zeros((1, hd1p), f32).at[0, :hd1].set(b1[0])

    w2_pad = jnp.zeros((hd1p, cp), f32).at[:hd1, :classes].set(w2).astype(bf16)
    a2sT = jnp.zeros((1, cp), f32).at[0, :classes].set(asrc2[0]).astype(bf16)
    a2d = jnp.zeros((cp, 1), f32).at[:classes, 0].set(adst2[0]).astype(bf16)
    b2p = jnp.zeros((1, cp), f32).at[0, :classes].set(b2[0])

    xp1, alsT, ald = pl.pallas_call(
        _proj_body,
        out_shape=(jax.ShapeDtypeStruct((np_, hd1p), bf16),
                   jax.ShapeDtypeStruct((hp, np_), f32),
                   jax.ShapeDtypeStruct((np_, hp), f32)),
        grid=grid,
        in_specs=[pl.BlockSpec((tm, fp), lambda i: (i, 0)),
                  _ANY_VMEM, _ANY_VMEM, _ANY_VMEM],
        out_specs=(pl.BlockSpec((tm, hd1p), lambda i: (i, 0)),
                   pl.BlockSpec((hp, tm), lambda i: (0, i)),
                   pl.BlockSpec((tm, hp), lambda i: (i, 0))),
        compiler_params=cparams,
    )(x_pad, w1_pad, c1T, a1d)

    xp2 = pl.pallas_call(
        functools.partial(_attn1_body, heads=heads, hidden=hidden, tile_m=tm),
        out_shape=jax.ShapeDtypeStruct((np_, cp), bf16),
        grid=grid,
        in_specs=[pl.BlockSpec((tm, np_), lambda i: (i, 0)),
                  _ANY_VMEM, _ANY_VMEM,
                  pl.BlockSpec((tm, hp), lambda i: (i, 0)),
                  _ANY_VMEM, _ANY_VMEM],
        out_specs=pl.BlockSpec((tm, cp), lambda i: (i, 0)),
        compiler_params=cparams,
    )(adj, xp1, alsT, ald, b1p, w2_pad)

    out = pl.pallas_call(
        functools.partial(_attn2_body, classes=classes, sf_num=sf_num,
                          tile_m=tm),
        out_shape=jax.ShapeDtypeStruct((np_, cp), f32),
        grid=grid,
        in_specs=[pl.BlockSpec((tm, np_), lambda i: (i, 0)),
                  _ANY_VMEM, _ANY_VMEM, _ANY_VMEM, _ANY_VMEM],
        out_specs=pl.BlockSpec((tm, cp), lambda i: (i, 0)),
        compiler_params=cparams,
    )(adj, xp2, a2sT, a2d, b2p)

    return out[:n, :sf_num], out[:n, sf_num:classes]


def kernel(x, edge_index, w1, asrc1, adst1, b1, w2, asrc2, adst2, b2):
    heads, hidden = asrc1.shape
    classes = w2.shape[1]
    return _gat(x, edge_index, w1, asrc1, adst1, b1, w2, asrc2, adst2, b2,
                heads=heads, hidden=hidden, classes=classes, sf_num=_SF_NUM,
                tile_m=256)
